# baseline (speedup 1.0000x reference)
import functools

import numpy as np
import jax
import jax.numpy as jnp
from jax import lax
from jax.experimental import pallas as pl
from jax.experimental.pallas import tpu as pltpu
from jax.experimental.pallas import tpu_sc as plsc

_NC, _NS, _LANES = 2, 16, 16
_NW = _NC * _NS
_CH = 32
_K = 4
_LOOK = 2


def _pe_table_np(seq_len: int, d_model: int) -> np.ndarray:
    pos = np.arange(seq_len, dtype=np.float64)[:, None]
    i = np.arange(d_model, dtype=np.float64)[None, :]
    angle_rates = np.power(10000.0, (2.0 * np.floor(i / 2.0)) / d_model)
    angles = pos / angle_rates
    even = (np.arange(d_model) % 2 == 0)
    pe = np.where(even[None, :], np.sin(angles), np.cos(angles))
    return pe.astype(np.float32)


@functools.cache
def _build(batch: int, seq_len: int, d: int):
    assert seq_len % _NW == 0 and d % _LANES == 0
    sp = seq_len // _NW
    assert sp % _CH == 0
    nchunk = sp // _CH
    ntask = batch * nchunk
    dgrp = d // _LANES

    mesh = plsc.VectorSubcoreMesh(
        core_axis_name="c", subcore_axis_name="s",
        num_cores=_NC, num_subcores=_NS)

    @functools.partial(
        pl.kernel,
        out_type=jax.ShapeDtypeStruct((batch * seq_len, d), jnp.float32),
        mesh=mesh,
        scratch_types=[
            pltpu.VMEM((batch * sp,), jnp.int32),
            pltpu.VMEM((sp, d), jnp.float32),
            pltpu.VMEM_SHARED((_NS * (sp // 2), d), jnp.float32),
            [pltpu.VMEM((_CH, d), jnp.float32) for _ in range(_K)],
            pltpu.SemaphoreType.DMA,
            pltpu.SemaphoreType.DMA,
            [pltpu.SemaphoreType.DMA for _ in range(_K)],
            [pltpu.SemaphoreType.DMA for _ in range(_K)],
        ],
    )
    def encode(idx_hbm, table_hbm, pe_hbm, out_hbm,
               idx_v, pe_v, pe_spm, bufs, isem, psem, gsems, osems):
        sub = lax.axis_index("s")
        wid = sub * _NC + lax.axis_index("c")
        s0 = wid * sp

        icopies = [
            pltpu.async_copy(idx_hbm.at[pl.ds(b * seq_len + s0, sp)],
                             idx_v.at[pl.ds(b * sp, sp)], isem)
            for b in range(batch)
        ]
        hp = sp // 2
        pe_stage = pltpu.async_copy(
            pe_hbm.at[pl.ds(s0, hp)], pe_spm.at[pl.ds(sub * hp, hp)], psem)
        for ic in icopies:
            ic.wait()

        def issue_gather(t):
            b, c = t // nchunk, t % nchunk
            return pltpu.async_copy(
                table_hbm.at[idx_v.at[pl.ds(b * sp + c * _CH, _CH)]],
                bufs[t % _K], gsems[t % _K])

        gathers = [None] * ntask
        writes = [None] * ntask
        for t in range(min(_LOOK, ntask)):
            gathers[t] = issue_gather(t)

        pe_stage.wait()
        pltpu.async_copy(
            pe_spm.at[pl.ds(sub * hp, hp)], pe_v.at[pl.ds(0, hp)],
            psem).wait()
        pltpu.async_copy(
            pe_hbm.at[pl.ds(s0 + hp, hp)], pe_spm.at[pl.ds(sub * hp, hp)],
            psem).wait()
        pltpu.async_copy(
            pe_spm.at[pl.ds(sub * hp, hp)], pe_v.at[pl.ds(hp, hp)],
            psem).wait()

        waited = [False] * ntask
        for t in range(ntask):
            gathers[t].wait()
            k = t % _K
            c = t % nchunk

            def add_row(r, carry):
                for g in range(dgrp):
                    sl = pl.ds(g * _LANES, _LANES)
                    plsc.addupdate(bufs[k].at[r, sl], pe_v[c * _CH + r, sl])
                return carry

            lax.fori_loop(0, _CH, add_row, 0)
            b = t // nchunk
            writes[t] = pltpu.async_copy(
                bufs[k],
                out_hbm.at[pl.ds(b * seq_len + s0 + c * _CH, _CH)],
                osems[k])
            nt = t + _LOOK
            if nt < ntask:
                prev = nt - _K
                if prev >= 0:
                    writes[prev].wait()
                    waited[prev] = True
                gathers[nt] = issue_gather(nt)
        for t in range(ntask):
            if not waited[t]:
                writes[t].wait()

    return encode


def kernel(input, embed_table):
    b, s = input.shape
    v, d = embed_table.shape
    idx = input.reshape(-1).astype(jnp.int32)
    pe = jnp.asarray(_pe_table_np(s, d))
    out = _build(b, s, d)(idx, embed_table, pe)
    return out.reshape(b, s, d)


# speedup vs baseline: 1.0198x; 1.0198x over previous
import functools

import numpy as np
import jax
import jax.numpy as jnp
from jax import lax
from jax.experimental import pallas as pl
from jax.experimental.pallas import tpu as pltpu
from jax.experimental.pallas import tpu_sc as plsc

_NC, _NS, _LANES = 2, 16, 16
_NW = _NC * _NS
_CH = 32
_K = 4
_LOOK = 2


def _pe_table_np(seq_len: int, d_model: int) -> np.ndarray:
    pos = np.arange(seq_len, dtype=np.float64)[:, None]
    i = np.arange(d_model, dtype=np.float64)[None, :]
    angle_rates = np.power(10000.0, (2.0 * np.floor(i / 2.0)) / d_model)
    angles = pos / angle_rates
    even = (np.arange(d_model) % 2 == 0)
    pe = np.where(even[None, :], np.sin(angles), np.cos(angles))
    return pe.astype(np.float32)


@functools.cache
def _build(batch: int, seq_len: int, d: int):
    assert seq_len % _NW == 0 and d % _LANES == 0
    sp = seq_len // _NW
    assert sp % _CH == 0
    nchunk = sp // _CH
    ntask = batch * nchunk
    dgrp = d // _LANES

    mesh = plsc.VectorSubcoreMesh(
        core_axis_name="c", subcore_axis_name="s",
        num_cores=_NC, num_subcores=_NS)

    @functools.partial(
        pl.kernel,
        out_type=jax.ShapeDtypeStruct((batch * seq_len, d), jnp.float32),
        mesh=mesh,
        scratch_types=[
            pltpu.VMEM((batch * sp,), jnp.int32),
            pltpu.VMEM((sp, d), jnp.float32),
            pltpu.VMEM_SHARED((_NS * (sp // 2), d), jnp.float32),
            [pltpu.VMEM((_CH, d), jnp.float32) for _ in range(_K)],
            pltpu.SemaphoreType.DMA,
            pltpu.SemaphoreType.DMA,
            [pltpu.SemaphoreType.DMA for _ in range(_K)],
            [pltpu.SemaphoreType.DMA for _ in range(_K)],
        ],
    )
    def encode(idx_hbm, table_hbm, pe_hbm, out_hbm,
               idx_v, pe_v, pe_spm, bufs, isem, psem, gsems, osems):
        sub = lax.axis_index("s")
        wid = sub * _NC + lax.axis_index("c")
        s0 = wid * sp

        icopies = [
            pltpu.async_copy(idx_hbm.at[pl.ds(b * seq_len + s0, sp)],
                             idx_v.at[pl.ds(b * sp, sp)], isem)
            for b in range(batch)
        ]
        pe_stage = pltpu.async_copy(pe_hbm.at[pl.ds(s0, sp)], pe_v, psem)
        for ic in icopies:
            ic.wait()

        def issue_gather(t):
            b, c = t // nchunk, t % nchunk
            return pltpu.async_copy(
                table_hbm.at[idx_v.at[pl.ds(b * sp + c * _CH, _CH)]],
                bufs[t % _K], gsems[t % _K])

        gathers = [None] * ntask
        writes = [None] * ntask
        for t in range(min(_LOOK, ntask)):
            gathers[t] = issue_gather(t)

        pe_stage.wait()

        waited = [False] * ntask
        for t in range(ntask):
            gathers[t].wait()
            k = t % _K
            c = t % nchunk

            def add_row(r, carry):
                for g in range(dgrp):
                    sl = pl.ds(g * _LANES, _LANES)
                    plsc.addupdate(bufs[k].at[r, sl], pe_v[c * _CH + r, sl])
                return carry

            lax.fori_loop(0, _CH, add_row, 0)
            b = t // nchunk
            writes[t] = pltpu.async_copy(
                bufs[k],
                out_hbm.at[pl.ds(b * seq_len + s0 + c * _CH, _CH)],
                osems[k])
            nt = t + _LOOK
            if nt < ntask:
                prev = nt - _K
                if prev >= 0:
                    writes[prev].wait()
                    waited[prev] = True
                gathers[nt] = issue_gather(nt)
        for t in range(ntask):
            if not waited[t]:
                writes[t].wait()

    return encode


def kernel(input, embed_table):
    b, s = input.shape
    v, d = embed_table.shape
    idx = input.reshape(-1).astype(jnp.int32)
    pe = jnp.asarray(_pe_table_np(s, d))
    out = _build(b, s, d)(idx, embed_table, pe)
    return out.reshape(b, s, d)


# speedup vs baseline: 1.1281x; 1.1062x over previous
import functools

import numpy as np
import jax
import jax.numpy as jnp
from jax import lax
from jax.experimental import pallas as pl
from jax.experimental.pallas import tpu as pltpu
from jax.experimental.pallas import tpu_sc as plsc

_NC, _NS, _LANES = 2, 16, 16
_NW = _NC * _NS
_CH = 16


def _pe_table_np(seq_len: int, d_model: int) -> np.ndarray:
    pos = np.arange(seq_len, dtype=np.float64)[:, None]
    i = np.arange(d_model, dtype=np.float64)[None, :]
    angle_rates = np.power(10000.0, (2.0 * np.floor(i / 2.0)) / d_model)
    angles = pos / angle_rates
    even = (np.arange(d_model) % 2 == 0)
    pe = np.where(even[None, :], np.sin(angles), np.cos(angles))
    return pe.astype(np.float32)


@functools.cache
def _build(batch: int, seq_len: int, d: int):
    assert seq_len % _NW == 0 and d % _LANES == 0
    sp = seq_len // _NW
    hp = sp // 2
    assert sp % _CH == 0 and hp % _CH == 0
    nchunk = sp // _CH
    dgrp = d // _LANES

    mesh = plsc.VectorSubcoreMesh(
        core_axis_name="c", subcore_axis_name="s",
        num_cores=_NC, num_subcores=_NS)

    @functools.partial(
        pl.kernel,
        out_type=jax.ShapeDtypeStruct((batch * seq_len, d), jnp.float32),
        mesh=mesh,
        scratch_types=[
            pltpu.VMEM((batch * sp,), jnp.int32),
            pltpu.VMEM((sp, d), jnp.float32),
            pltpu.VMEM_SHARED((_NS * hp, d), jnp.float32),
            [[pltpu.VMEM((_CH, d), jnp.float32) for _ in range(2)]
             for _ in range(batch)],
            pltpu.SemaphoreType.DMA,
            pltpu.SemaphoreType.DMA,
            [pltpu.SemaphoreType.DMA for _ in range(2)],
            [pltpu.SemaphoreType.DMA for _ in range(2)],
        ],
    )
    def encode(idx_hbm, table_hbm, pe_hbm, out_hbm,
               idx_v, pe_v, pe_spm, bufs, isem, psem, gsems, osems):
        sub = lax.axis_index("s")
        wid = sub * _NC + lax.axis_index("c")
        s0 = wid * sp

        icopies = [
            pltpu.async_copy(idx_hbm.at[pl.ds(b * seq_len + s0, sp)],
                             idx_v.at[pl.ds(b * sp, sp)], isem)
            for b in range(batch)
        ]
        pe_h0 = pltpu.async_copy(
            pe_hbm.at[pl.ds(s0, hp)], pe_spm.at[pl.ds(sub * hp, hp)], psem)
        for ic in icopies:
            ic.wait()

        def gathers(c):
            par = c % 2
            return [
                pltpu.async_copy(
                    table_hbm.at[idx_v.at[pl.ds(b * sp + c * _CH, _CH)]],
                    bufs[b][par], gsems[par])
                for b in range(batch)
            ]

        pend = gathers(0)

        pe_h0.wait()
        pltpu.async_copy(
            pe_spm.at[pl.ds(sub * hp, hp)], pe_v.at[pl.ds(0, hp)],
            psem).wait()
        pe_h1 = pltpu.async_copy(
            pe_hbm.at[pl.ds(s0 + hp, hp)], pe_spm.at[pl.ds(sub * hp, hp)],
            psem)
        pe_stage1 = None

        owrites = [None, None]
        for c in range(nchunk):
            par = c % 2
            if (c + 1) * _CH == hp:
                pe_h1.wait()
                pe_stage1 = pltpu.async_copy(
                    pe_spm.at[pl.ds(sub * hp, hp)], pe_v.at[pl.ds(hp, hp)],
                    psem)
            if c * _CH == hp:
                pe_stage1.wait()
            for g in pend:
                g.wait()
            if c + 1 < nchunk:
                if owrites[1 - par] is not None:
                    for w in owrites[1 - par]:
                        w.wait()
                    owrites[1 - par] = None
                pend = gathers(c + 1)

            def add_row(r, carry):
                for k in range(dgrp):
                    sl = pl.ds(k * _LANES, _LANES)
                    v = pe_v[c * _CH + r, sl]
                    for b in range(batch):
                        plsc.addupdate(bufs[b][par].at[r, sl], v)
                return carry

            lax.fori_loop(0, _CH, add_row, 0)
            owrites[par] = [
                pltpu.async_copy(
                    bufs[b][par],
                    out_hbm.at[pl.ds(b * seq_len + s0 + c * _CH, _CH)],
                    osems[par])
                for b in range(batch)
            ]
        for ws in owrites:
            if ws is not None:
                for w in ws:
                    w.wait()

    return encode


def kernel(input, embed_table):
    b, s = input.shape
    v, d = embed_table.shape
    idx = input.reshape(-1).astype(jnp.int32)
    pe = jnp.asarray(_pe_table_np(s, d))
    out = _build(b, s, d)(idx, embed_table, pe)
    return out.reshape(b, s, d)
